# Initial kernel scaffold; baseline (speedup 1.0000x reference)
import jax
import jax.numpy as jnp
from jax.experimental import pallas as pl


def kernel(inputs, tables):
    raise NotImplementedError("write your pallas kernel here")


# speedup vs baseline: 1.2987x; 1.2987x over previous
import functools

import jax
import jax.numpy as jnp
from jax import lax
from jax.experimental import pallas as pl
from jax.experimental.pallas import tpu as pltpu
from jax.experimental.pallas import tpu_sc as plsc

_CHUNK = 128
_LANES = 16


@functools.lru_cache(maxsize=None)
def _build(B, F, V, D):
    info = plsc.get_sparse_core_info()
    NC, NS = info.num_cores, info.num_subcores
    NW = NC * NS
    total = B * F
    assert total % NW == 0
    per_w = total // NW
    assert per_w % _CHUNK == 0
    n_chunks = per_w // _CHUNK
    vecs_per_chunk = _CHUNK // _LANES

    mesh = plsc.VectorSubcoreMesh(core_axis_name="c", subcore_axis_name="s")

    @functools.partial(
        pl.kernel,
        mesh=mesh,
        out_type=jax.ShapeDtypeStruct((total,), jnp.float32),
        scratch_types=[
            pltpu.VMEM((per_w,), jnp.float32),
            pltpu.VMEM((n_chunks, _CHUNK), jnp.int32),
            pltpu.VMEM((per_w,), jnp.float32),
            pltpu.SemaphoreType.DMA,
        ],
    )
    def k(tab_hbm, in_hbm, out_hbm, in_v, idx_v, got_v, sem):
        wid = lax.axis_index("s") * NC + lax.axis_index("c")
        base = wid * per_w
        pltpu.sync_copy(in_hbm.at[pl.ds(base, per_w)], in_v)

        lane = lax.iota(jnp.int32, _LANES)
        copies = []
        for j in range(n_chunks):
            def step(i, _):
                off = j * _CHUNK + i * _LANES
                ids = in_v[pl.ds(off, _LANES)].astype(jnp.int32)
                p = base + off + lane
                f = lax.rem(p, F)
                idx_v[j, pl.ds(i * _LANES, _LANES)] = (f * V + ids) * D
                return 0

            lax.fori_loop(0, vecs_per_chunk, step, 0)
            copies.append(
                pltpu.async_copy(
                    tab_hbm.at[idx_v.at[j]],
                    got_v.at[pl.ds(j * _CHUNK, _CHUNK)],
                    sem,
                )
            )
        for c in copies:
            c.wait()
        pltpu.sync_copy(got_v, out_hbm.at[pl.ds(base, per_w)])

    return k


def kernel(inputs, tables):
    B, F = inputs.shape
    Ft, V, D = tables.shape
    out_flat = _build(B, F, V, D)(tables.reshape(-1), inputs.reshape(-1))
    return out_flat.reshape(B, F)


# speedup vs baseline: 2.0575x; 1.5843x over previous
import functools

import jax
import jax.numpy as jnp
from jax import lax
from jax.experimental import pallas as pl
from jax.experimental.pallas import tpu as pltpu
from jax.experimental.pallas import tpu_sc as plsc

_CHUNK = 128
_LANES = 16


_SLAB = 256


@functools.lru_cache(maxsize=None)
def _build_compress(R, D):
    info = plsc.get_sparse_core_info()
    NW = info.num_cores * info.num_subcores
    n_slabs = -(-R // (_SLAB * NW))
    n_slabs += n_slabs % 2
    assert R % 8 == 0 and (R - _SLAB) % 8 == 0 and n_slabs >= 2

    mesh = plsc.VectorSubcoreMesh(core_axis_name="c", subcore_axis_name="s")

    @functools.partial(
        pl.kernel,
        mesh=mesh,
        compiler_params=pltpu.CompilerParams(needs_layout_passes=False),
        out_type=jax.ShapeDtypeStruct((R,), jnp.float32),
        scratch_types=[
            pltpu.VMEM((_SLAB, D), jnp.float32),
            pltpu.VMEM((_SLAB, D), jnp.float32),
            pltpu.VMEM((_SLAB,), jnp.float32),
            pltpu.VMEM((_SLAB,), jnp.float32),
            pltpu.SemaphoreType.DMA,
            pltpu.SemaphoreType.DMA,
        ],
    )
    def k(tab_hbm, col_hbm, slab0, slab1, col0, col1, rsem, wsem):
        wid = lax.axis_index("s") * info.num_cores + lax.axis_index("c")
        slabs = (slab0, slab1)
        cols = (col0, col1)
        lane = lax.iota(jnp.int32, _LANES)

        def slab_start(i):
            start = jnp.minimum((wid + i * NW) * _SLAB, R - _SLAB)
            return pl.multiple_of(start, 8)

        def rd(i, b):
            return pltpu.make_async_copy(
                tab_hbm.at[pl.ds(slab_start(i), _SLAB)], slabs[b], rsem
            )

        def wr(i, b):
            return pltpu.make_async_copy(
                cols[b], col_hbm.at[pl.ds(slab_start(i), _SLAB)], wsem
            )

        rd(0, 0).start()
        rd(1, 1).start()

        @pl.loop(0, n_slabs, step=2)
        def body(g):
            for b in range(2):
                i = g + b
                rd(i, b).wait()

                @pl.when(i >= 2)
                def _():
                    wr(i - 2, b).wait()

                def extract(v, _):
                    vals = plsc.load_gather(
                        slabs[b], [v * _LANES + lane, lane * 0]
                    )
                    cols[b][pl.ds(v * _LANES, _LANES)] = vals
                    return 0

                lax.fori_loop(0, _SLAB // _LANES, extract, 0)

                @pl.when(i + 2 < n_slabs)
                def _():
                    rd(i + 2, b).start()

                wr(i, b).start()

        wr(n_slabs - 2, 0).wait()
        wr(n_slabs - 1, 1).wait()

    return k


@functools.lru_cache(maxsize=None)
def _build_gather(B, F, V):
    total = B * F
    info = plsc.get_sparse_core_info()
    NC, NS = info.num_cores, info.num_subcores
    NW = NC * NS
    assert total % NW == 0
    per_w = total // NW
    assert per_w % _CHUNK == 0
    n_chunks = per_w // _CHUNK
    vecs_per_chunk = _CHUNK // _LANES

    mesh = plsc.VectorSubcoreMesh(core_axis_name="c", subcore_axis_name="s")

    @functools.partial(
        pl.kernel,
        mesh=mesh,
        out_type=jax.ShapeDtypeStruct((total,), jnp.float32),
        scratch_types=[
            pltpu.VMEM((per_w,), jnp.float32),
            pltpu.VMEM((n_chunks, _CHUNK), jnp.int32),
            pltpu.VMEM((per_w,), jnp.float32),
            pltpu.SemaphoreType.DMA,
        ],
    )
    def k(col_hbm, in_hbm, out_hbm, in_v, idx_v, got_v, sem):
        wid = lax.axis_index("s") * NC + lax.axis_index("c")
        base = wid * per_w
        pltpu.sync_copy(in_hbm.at[pl.ds(base, per_w)], in_v)

        lane = lax.iota(jnp.int32, _LANES)
        copies = []
        for j in range(n_chunks):
            def step(i, _):
                off = j * _CHUNK + i * _LANES
                ids = in_v[pl.ds(off, _LANES)].astype(jnp.int32)
                p = base + off + lane
                f = lax.rem(p, F)
                idx_v[j, pl.ds(i * _LANES, _LANES)] = f * V + ids
                return 0

            lax.fori_loop(0, vecs_per_chunk, step, 0)
            copies.append(
                pltpu.async_copy(
                    col_hbm.at[idx_v.at[j]],
                    got_v.at[pl.ds(j * _CHUNK, _CHUNK)],
                    sem,
                )
            )
        for c in copies:
            c.wait()
        pltpu.sync_copy(got_v, out_hbm.at[pl.ds(base, per_w)])

    return k


def kernel(inputs, tables):
    B, F = inputs.shape
    Ft, V, D = tables.shape
    col0 = _build_compress(Ft * V, D)(tables.reshape(Ft * V, D))
    out_flat = _build_gather(B, F, V)(col0, inputs.reshape(-1))
    return out_flat.reshape(B, F)


# speedup vs baseline: 3.0451x; 1.4800x over previous
import functools

import jax
import jax.numpy as jnp
from jax import lax
from jax.experimental import pallas as pl
from jax.experimental.pallas import tpu as pltpu
from jax.experimental.pallas import tpu_sc as plsc

_RING = 16
_LANES = 16


@functools.lru_cache(maxsize=None)
def _build(B, F, V, D):
    info = plsc.get_sparse_core_info()
    NC, NS = info.num_cores, info.num_subcores
    NW = NC * NS
    total = B * F
    assert total % NW == 0
    per_w = total // NW
    assert per_w % _RING == 0 and per_w % _LANES == 0 and V % 8 == 0

    mesh = plsc.VectorSubcoreMesh(core_axis_name="c", subcore_axis_name="s")

    @functools.partial(
        pl.kernel,
        mesh=mesh,
        compiler_params=pltpu.CompilerParams(needs_layout_passes=False),
        out_type=jax.ShapeDtypeStruct((total,), jnp.float32),
        scratch_types=[
            pltpu.VMEM((per_w,), jnp.float32),
            pltpu.VMEM((per_w,), jnp.int32),
            pltpu.VMEM((per_w,), jnp.int32),
            pltpu.VMEM((per_w,), jnp.float32),
            *([pltpu.VMEM((8, D), jnp.float32)] * _RING),
            pltpu.SemaphoreType.DMA,
        ],
    )
    def k(tab_hbm, in_hbm, out_hbm, in_v, row_v, sub_v, got_v, *ring_sem):
        ring = ring_sem[:_RING]
        sem = ring_sem[_RING]
        wid = lax.axis_index("s") * NC + lax.axis_index("c")
        base = wid * per_w
        pltpu.sync_copy(in_hbm.at[pl.ds(base, per_w)], in_v)

        lane = lax.iota(jnp.int32, _LANES)

        def index(i, _):
            off = i * _LANES
            ids = in_v[pl.ds(off, _LANES)].astype(jnp.int32)
            p = base + off + lane
            f = lax.rem(p, F)
            r = f * V + ids
            row_v[pl.ds(off, _LANES)] = r - lax.bitwise_and(r, 7)
            sub_v[pl.ds(off, _LANES)] = lax.bitwise_and(r, 7)
            return 0

        lax.fori_loop(0, per_w // _LANES, index, 0)

        def dma(start, b):
            start = pl.multiple_of(start, 8)
            return pltpu.make_async_copy(
                tab_hbm.at[pl.ds(start, 8)], ring[b], sem
            )

        rows0 = row_v[pl.ds(0, _LANES)]
        for b in range(_RING):
            dma(rows0[b], b).start()

        zero = lane * 0
        fzero = lane * 0.0

        @pl.loop(0, per_w, step=_RING)
        def body(g):
            rows = row_v[pl.ds(g, _LANES)]
            subs = sub_v[pl.ds(g, _LANES)]
            outv = fzero
            for b in range(_RING):
                dma(rows[b], b).wait()
                vals = plsc.load_gather(ring[b], [zero + subs[b], zero])
                outv = jnp.where(lane == b, vals, outv)

                @pl.when(g + b + _RING < per_w)
                def _():
                    nrows = row_v[pl.ds(g + _RING, _LANES)]
                    dma(nrows[b], b).start()

            got_v[pl.ds(g, _LANES)] = outv

        pltpu.sync_copy(got_v, out_hbm.at[pl.ds(base, per_w)])

    return k


def kernel(inputs, tables):
    B, F = inputs.shape
    Ft, V, D = tables.shape
    out_flat = _build(B, F, V, D)(
        tables.reshape(Ft * V, D), inputs.reshape(-1)
    )
    return out_flat.reshape(B, F)


# speedup vs baseline: 3.1585x; 1.0373x over previous
import functools

import jax
import jax.numpy as jnp
from jax import lax
from jax.experimental import pallas as pl
from jax.experimental.pallas import tpu as pltpu
from jax.experimental.pallas import tpu_sc as plsc

_RING = 16
_LANES = 16


@functools.lru_cache(maxsize=None)
def _build(B, F, V, D):
    info = plsc.get_sparse_core_info()
    NC, NS = info.num_cores, info.num_subcores
    NW = NC * NS
    total = B * F
    assert total % NW == 0
    per_w = total // NW
    assert per_w % (2 * _LANES) == 0 and V % 8 == 0

    mesh = plsc.VectorSubcoreMesh(core_axis_name="c", subcore_axis_name="s")

    @functools.partial(
        pl.kernel,
        mesh=mesh,
        compiler_params=pltpu.CompilerParams(needs_layout_passes=False),
        out_type=jax.ShapeDtypeStruct((total,), jnp.float32),
        scratch_types=[
            pltpu.VMEM((per_w,), jnp.float32),
            pltpu.VMEM((per_w,), jnp.int32),
            pltpu.VMEM((per_w,), jnp.int32),
            pltpu.VMEM((per_w,), jnp.float32),
            pltpu.VMEM((_LANES, 8, D), jnp.float32),
            pltpu.VMEM((_LANES, 8, D), jnp.float32),
            pltpu.SemaphoreType.DMA,
        ],
    )
    def k(tab_hbm, in_hbm, out_hbm, in_v, row_v, sub_v, got_v, h0, h1, sem):
        ring = (h0, h1)
        wid = lax.axis_index("s") * NC + lax.axis_index("c")
        base = wid * per_w
        pltpu.sync_copy(in_hbm.at[pl.ds(base, per_w)], in_v)

        lane = lax.iota(jnp.int32, _LANES)

        def index(i, _):
            off = i * _LANES
            ids = in_v[pl.ds(off, _LANES)].astype(jnp.int32)
            p = base + off + lane
            f = lax.rem(p, F)
            r = f * V + ids
            row_v[pl.ds(off, _LANES)] = r - lax.bitwise_and(r, 7)
            sub_v[pl.ds(off, _LANES)] = lax.bitwise_and(r, 7)
            return 0

        lax.fori_loop(0, per_w // _LANES, index, 0)

        def dma(start, h, l):
            start = pl.multiple_of(start, 8)
            return pltpu.make_async_copy(
                tab_hbm.at[pl.ds(start, 8)], ring[h].at[l], sem
            )

        for h in range(2):
            rows0 = row_v[pl.ds(h * _LANES, _LANES)]
            for l in range(_LANES):
                dma(rows0[l], h, l).start()

        zero = lane * 0
        step = 2 * _LANES

        @pl.loop(0, per_w, step=step)
        def body(g):
            for h in range(2):
                off = g + h * _LANES
                rows = row_v[pl.ds(off, _LANES)]
                subs = sub_v[pl.ds(off, _LANES)]
                for l in range(_LANES):
                    dma(rows[l], h, l).wait()
                got_v[pl.ds(off, _LANES)] = plsc.load_gather(
                    ring[h], [lane, subs, zero]
                )

                @pl.when(g + step < per_w)
                def _():
                    nrows = row_v[pl.ds(off + step, _LANES)]
                    for l in range(_LANES):
                        dma(nrows[l], h, l).start()

        pltpu.sync_copy(got_v, out_hbm.at[pl.ds(base, per_w)])

    return k


def kernel(inputs, tables):
    B, F = inputs.shape
    Ft, V, D = tables.shape
    out_flat = _build(B, F, V, D)(
        tables.reshape(Ft * V, D), inputs.reshape(-1)
    )
    return out_flat.reshape(B, F)


# speedup vs baseline: 3.1825x; 1.0076x over previous
import functools

import jax
import jax.numpy as jnp
from jax import lax
from jax.experimental import pallas as pl
from jax.experimental.pallas import tpu as pltpu
from jax.experimental.pallas import tpu_sc as plsc

_RING = 16
_LANES = 16


@functools.lru_cache(maxsize=None)
def _build(B, F, V, D):
    info = plsc.get_sparse_core_info()
    NC, NS = info.num_cores, info.num_subcores
    NW = NC * NS
    total = B * F
    assert total % NW == 0
    per_w = total // NW
    assert per_w % (2 * _LANES) == 0 and V % 8 == 0

    mesh = plsc.VectorSubcoreMesh(core_axis_name="c", subcore_axis_name="s")

    @functools.partial(
        pl.kernel,
        mesh=mesh,
        compiler_params=pltpu.CompilerParams(needs_layout_passes=False),
        out_type=jax.ShapeDtypeStruct((total,), jnp.float32),
        scratch_types=[
            pltpu.VMEM((per_w,), jnp.float32),
            pltpu.VMEM((per_w,), jnp.int32),
            pltpu.VMEM((per_w,), jnp.int32),
            pltpu.VMEM((per_w,), jnp.float32),
            pltpu.VMEM((_LANES, 8, D), jnp.float32),
            pltpu.VMEM((_LANES, 8, D), jnp.float32),
            pltpu.SemaphoreType.DMA,
        ],
    )
    def k(tab_hbm, in_hbm, out_hbm, in_v, row_v, sub_v, got_v, h0, h1, sem):
        ring = (h0, h1)
        wid = lax.axis_index("s") * NC + lax.axis_index("c")
        base = wid * per_w
        pltpu.sync_copy(in_hbm.at[pl.ds(base, per_w)], in_v)

        lane = lax.iota(jnp.int32, _LANES)

        def index(i, _):
            off = i * _LANES
            ids = in_v[pl.ds(off, _LANES)].astype(jnp.int32)
            p = base + off + lane
            f = lax.rem(p, F)
            r = f * V + ids
            row_v[pl.ds(off, _LANES)] = lax.shift_right_logical(r, 3)
            sub_v[pl.ds(off, _LANES)] = lax.bitwise_and(r, 7)
            return 0

        lax.fori_loop(0, per_w // _LANES, index, 0)

        def dma(tile, h, l):
            return pltpu.make_async_copy(
                tab_hbm.at[tile], ring[h].at[l], sem
            )

        for h in range(2):
            rows0 = row_v[pl.ds(h * _LANES, _LANES)]
            for l in range(_LANES):
                dma(rows0[l], h, l).start()

        zero = lane * 0
        step = 2 * _LANES

        @pl.loop(0, per_w, step=step)
        def body(g):
            for h in range(2):
                off = g + h * _LANES
                rows = row_v[pl.ds(off, _LANES)]
                subs = sub_v[pl.ds(off, _LANES)]
                for l in range(_LANES):
                    dma(rows[l], h, l).wait()
                got_v[pl.ds(off, _LANES)] = plsc.load_gather(
                    ring[h], [lane, subs, zero]
                )

                @pl.when(g + step < per_w)
                def _():
                    nrows = row_v[pl.ds(off + step, _LANES)]
                    for l in range(_LANES):
                        dma(nrows[l], h, l).start()

        pltpu.sync_copy(got_v, out_hbm.at[pl.ds(base, per_w)])

    return k


def kernel(inputs, tables):
    B, F = inputs.shape
    Ft, V, D = tables.shape
    out_flat = _build(B, F, V, D)(
        tables.reshape(Ft * V // 8, 8, D), inputs.reshape(-1)
    )
    return out_flat.reshape(B, F)
